# baseline (speedup 1.0000x reference)
import functools

import jax
import jax.numpy as jnp
from jax import lax
from jax.experimental import pallas as pl
from jax.experimental.pallas import tpu as pltpu
from jax.experimental.pallas import tpu_sc as plsc

VOCAB = 1000000
MAXLEN = 200
EMBED_DIM = 64
BATCH = 4096
SEQ = 200

NUM_WORKERS = 32
ROWS_PER_WORKER = (BATCH * SEQ) // NUM_WORKERS
SEQS_PER_WORKER = ROWS_PER_WORKER // SEQ


def _tec_body(x_hbm, tok_hbm, pos_hbm, out_hbm, idx_v, rows_v, pos_v, sem):
    wid = lax.axis_index("s") * 2 + lax.axis_index("c")
    base = wid * ROWS_PER_WORKER

    pltpu.sync_copy(pos_hbm, pos_v)

    def seq_step(i, carry):
        row0 = base + i * SEQ
        pltpu.sync_copy(x_hbm.at[pl.ds(row0, SEQ)], idx_v)
        pltpu.async_copy(tok_hbm.at[idx_v], rows_v, sem).wait()

        def add_row(r, c):
            for j in range(EMBED_DIM // 16):
                sl = pl.ds(j * 16, 16)
                rows_v[r, sl] = rows_v[r, sl] + pos_v[r, sl]
            return c

        lax.fori_loop(0, SEQ, add_row, 0, unroll=4)
        pltpu.sync_copy(rows_v, out_hbm.at[pl.ds(row0, SEQ)])
        return carry

    lax.fori_loop(0, SEQS_PER_WORKER, seq_step, 0)


@jax.jit
def _embed(x_flat, token_table, pos_table):
    mesh = plsc.VectorSubcoreMesh(core_axis_name="c", subcore_axis_name="s")
    return pl.kernel(
        _tec_body,
        out_type=jax.ShapeDtypeStruct((BATCH * SEQ, EMBED_DIM), jnp.float32),
        mesh=mesh,
        scratch_types=[
            pltpu.VMEM((SEQ,), jnp.int32),
            pltpu.VMEM((SEQ, EMBED_DIM), jnp.float32),
            pltpu.VMEM((MAXLEN, EMBED_DIM), jnp.float32),
            pltpu.SemaphoreType.DMA,
        ],
        compiler_params=pltpu.CompilerParams(use_tc_tiling_on_sc=False),
    )(x_flat, token_table, pos_table)


def kernel(x, token_table, pos_table):
    out = _embed(x.reshape(-1), token_table, pos_table)
    return out.reshape(BATCH, SEQ, EMBED_DIM)


# speedup vs baseline: 1.1411x; 1.1411x over previous
import jax
import jax.numpy as jnp
from jax import lax
from jax.experimental import pallas as pl
from jax.experimental.pallas import tpu as pltpu
from jax.experimental.pallas import tpu_sc as plsc

VOCAB = 1000000
MAXLEN = 200
EMBED_DIM = 64
BATCH = 4096
SEQ = 200

NUM_WORKERS = 32
ROWS_PER_WORKER = (BATCH * SEQ) // NUM_WORKERS
CHUNK = 128
CHUNKS_PER_WORKER = ROWS_PER_WORKER // CHUNK
NBUF = 4
OUTER = CHUNKS_PER_WORKER // NBUF


def _tec_body(x_hbm, tok_hbm, pos_hbm, out_hbm, idx_v, pos2_v, g_bufs, o_bufs,
              g_sems, o_sems):
    wid = lax.axis_index("s") * 2 + lax.axis_index("c")
    base = wid * ROWS_PER_WORKER

    pltpu.sync_copy(x_hbm.at[pl.ds(wid * CHUNKS_PER_WORKER, CHUNKS_PER_WORKER)],
                    idx_v)
    pltpu.sync_copy(pos_hbm, pos2_v.at[pl.ds(0, MAXLEN)])
    pltpu.sync_copy(pos_hbm, pos2_v.at[pl.ds(MAXLEN, MAXLEN)])

    def gather_start(c, b):
        pltpu.make_async_copy(tok_hbm.at[idx_v.at[c]], g_bufs[b],
                              g_sems.at[b]).start()

    def gather_wait(c, b):
        pltpu.make_async_copy(tok_hbm.at[idx_v.at[c]], g_bufs[b],
                              g_sems.at[b]).wait()

    def out_copy(c, b):
        return pltpu.make_async_copy(
            o_bufs[b], out_hbm.at[pl.ds(base + c * CHUNK, CHUNK)], o_sems.at[b])

    for b in range(NBUF):
        gather_start(b, b)

    def outer_step(m, carry):
        for b in range(NBUF):
            c = m * NBUF + b
            gather_wait(c, b)

            g = g_bufs[b]
            o = o_bufs[b]
            p0 = lax.rem(c * CHUNK, MAXLEN)

            def add_row(k, _):
                for j in range(EMBED_DIM // 16):
                    sl = pl.ds(j * 16, 16)
                    o[k, sl] = g[k, sl] + pos2_v[p0 + k, sl]
                return _

            lax.fori_loop(0, CHUNK, add_row, 0, unroll=8)

            gather_start(lax.rem(c + NBUF, CHUNKS_PER_WORKER), b)

            @pl.when(m > 0)
            def _():
                out_copy(c, b).wait()

            out_copy(c, b).start()
        return carry

    lax.fori_loop(0, OUTER, outer_step, 0)

    for b in range(NBUF):
        gather_wait(b, b)
        out_copy(CHUNKS_PER_WORKER - NBUF + b, b).wait()


@jax.jit
def _embed(x2d, token_table, pos_table):
    mesh = plsc.VectorSubcoreMesh(core_axis_name="c", subcore_axis_name="s")
    return pl.kernel(
        _tec_body,
        out_type=jax.ShapeDtypeStruct((BATCH * SEQ, EMBED_DIM), jnp.float32),
        mesh=mesh,
        scratch_types=[
            pltpu.VMEM((CHUNKS_PER_WORKER, CHUNK), jnp.int32),
            pltpu.VMEM((2 * MAXLEN, EMBED_DIM), jnp.float32),
            [pltpu.VMEM((CHUNK, EMBED_DIM), jnp.float32) for _ in range(NBUF)],
            [pltpu.VMEM((CHUNK, EMBED_DIM), jnp.float32) for _ in range(NBUF)],
            pltpu.SemaphoreType.DMA((NBUF,)),
            pltpu.SemaphoreType.DMA((NBUF,)),
        ],
        compiler_params=pltpu.CompilerParams(use_tc_tiling_on_sc=False),
    )(x2d, token_table, pos_table)


def kernel(x, token_table, pos_table):
    out = _embed(x.reshape(-1, CHUNK), token_table, pos_table)
    return out.reshape(BATCH, SEQ, EMBED_DIM)
